# Initial kernel scaffold; baseline (speedup 1.0000x reference)
import jax
import jax.numpy as jnp
from jax.experimental import pallas as pl


def kernel(x, mask, ratio, ln_w, ln_b, W_inproj, W_tc, W_cc1, W_cc2, W_cc3, W_ch, W_c3, alpha):
    raise NotImplementedError("write your pallas kernel here")


# speedup vs baseline: 1.0644x; 1.0644x over previous
import jax
import jax.numpy as jnp
from jax.experimental import pallas as pl
from jax.experimental.pallas import tpu as pltpu

_INV_SQRT2 = 0.7071067811865476


def _gelu_exact(v):
    return 0.5 * v * (1.0 + jax.lax.erf(v * _INV_SQRT2))


def _rt(v):
    return v.astype(jnp.bfloat16).astype(jnp.float32)


def _rt_hard(v):
    b = jax.lax.bitcast_convert_type(v, jnp.uint32)
    r = b + jnp.uint32(0x7FFF) + ((b >> 16) & jnp.uint32(1))
    return jax.lax.bitcast_convert_type(r & jnp.uint32(0xFFFF0000),
                                        jnp.float32)


def _ln_inproj(x_ref, mu_ref, sig_ref, lnw_ref, lnb_ref, wipT_ref):
    xt = x_ref[0].T
    mu = mu_ref[0].T
    sig = sig_ref[0].T
    xn = (xt - mu) / jnp.sqrt(sig + 1e-5) * lnw_ref[...] + lnb_ref[...]
    local = _gelu_exact(
        jax.lax.dot(xn.astype(jnp.bfloat16), wipT_ref[...],
                    preferred_element_type=jnp.float32))
    return local


def _passA_body(x_ref, mu_ref, sig_ref, m_ref, lnw_ref, lnb_ref, wipT_ref,
                num_ref, den_ref):
    j = pl.program_id(1)
    local = _ln_inproj(x_ref, mu_ref, sig_ref, lnw_ref, lnb_ref, wipT_ref)
    mt = m_ref[0].T
    num_p = jnp.sum(local * mt, axis=0, keepdims=True)
    den_p = jnp.sum(mt)

    @pl.when(j == 0)
    def _():
        num_ref[...] = jnp.zeros_like(num_ref)
        den_ref[...] = jnp.zeros_like(den_ref)

    num_ref[...] += num_p[None]
    den_ref[...] += den_p.reshape(1, 1, 1)


def _passB_body(x_ref, mu_ref, sig_ref, lnw_ref, lnb_ref, wipT_ref,
                s1_ref, y2_ref, gx_ref, wcc1T_ref, wcc2T_ref, wcc3T_ref,
                wch_ref, wc3_ref, sa_ref, out_ref, wgt_ref):
    blk = x_ref.shape[2]
    C = x_ref.shape[1]
    local = _ln_inproj(x_ref, mu_ref, sig_ref, lnw_ref, lnb_ref, wipT_ref)
    gx = gx_ref[0]
    t = local * gx
    pmax = jnp.max(t, axis=-1, keepdims=True)
    pmean = t.mean(-1, keepdims=True)
    y1 = (local * s1_ref[0]).astype(jnp.bfloat16)
    y2 = jnp.broadcast_to(y2_ref[0].astype(jnp.bfloat16), (blk, C))
    y = jnp.concatenate([y1, y2], axis=1)
    z1 = _gelu_exact(jax.lax.dot(y, wcc1T_ref[...],
                                 preferred_element_type=jnp.float32))
    z2 = _gelu_exact(jax.lax.dot(z1.astype(jnp.bfloat16), wcc2T_ref[...],
                                 preferred_element_type=jnp.float32))
    z3 = _gelu_exact(jax.lax.dot(z2.astype(jnp.bfloat16), wcc3T_ref[...],
                                 preferred_element_type=jnp.float32))
    feat = jnp.concatenate([pmax, pmean, z3], axis=1)
    ft = feat.T
    x1_0 = _gelu_exact(wch_ref[0, 0] * _rt(ft[0:1]) + wch_ref[0, 1] * _rt(ft[1:2]))
    x1_1 = _gelu_exact(wch_ref[1, 0] * _rt(ft[0:1]) + wch_ref[1, 1] * _rt(ft[1:2]))
    pre = (wc3_ref[0, 0] * _rt(x1_0) + wc3_ref[0, 1] * _rt(x1_1)
           + wc3_ref[0, 2] * _rt(ft[2:3]) + wc3_ref[0, 3] * _rt(ft[3:4]))
    out = _gelu_exact(pre)
    out_ref[...] = out[None]
    wgt_ref[...] = (sa_ref[0, 0] * _gelu_exact(out))[None]


def _run_passA(x3, mu2, sig2, mask2, lnw, lnb, wipT_bf, blk):
    B, C, HW = x3.shape
    nb = HW // blk
    num, den = pl.pallas_call(
        _passA_body,
        grid=(B, nb),
        in_specs=[
            pl.BlockSpec((1, C, blk), lambda b, j: (b, 0, j)),
            pl.BlockSpec((1, 1, blk), lambda b, j: (b, 0, j)),
            pl.BlockSpec((1, 1, blk), lambda b, j: (b, 0, j)),
            pl.BlockSpec((1, 1, blk), lambda b, j: (b, 0, j)),
            pl.BlockSpec((1, C), lambda b, j: (0, 0)),
            pl.BlockSpec((1, C), lambda b, j: (0, 0)),
            pl.BlockSpec((C, C), lambda b, j: (0, 0)),
        ],
        out_specs=[
            pl.BlockSpec((1, 1, C), lambda b, j: (b, 0, 0)),
            pl.BlockSpec((1, 1, 1), lambda b, j: (b, 0, 0)),
        ],
        out_shape=[
            jax.ShapeDtypeStruct((B, 1, C), jnp.float32),
            jax.ShapeDtypeStruct((B, 1, 1), jnp.float32),
        ],
        compiler_params=pltpu.CompilerParams(
            dimension_semantics=("arbitrary", "arbitrary")),
    )(x3, mu2, sig2, mask2, lnw, lnb, wipT_bf)
    return num, den


def _run_passB(x3, mu2, sig2, lnw, lnb, wipT_bf, s1c, y2c, gxc, wcc1T_bf,
               wcc2T_bf, wcc3T_bf, wch_r, wc3_r, sig_a, blk):
    B, C, HW = x3.shape
    nb = HW // blk
    out, wgt = pl.pallas_call(
        _passB_body,
        grid=(B, nb),
        in_specs=[
            pl.BlockSpec((1, C, blk), lambda b, j: (b, 0, j)),
            pl.BlockSpec((1, 1, blk), lambda b, j: (b, 0, j)),
            pl.BlockSpec((1, 1, blk), lambda b, j: (b, 0, j)),
            pl.BlockSpec((1, C), lambda b, j: (0, 0)),
            pl.BlockSpec((1, C), lambda b, j: (0, 0)),
            pl.BlockSpec((C, C), lambda b, j: (0, 0)),
            pl.BlockSpec((1, 1, C), lambda b, j: (b, 0, 0)),
            pl.BlockSpec((1, 1, C), lambda b, j: (b, 0, 0)),
            pl.BlockSpec((1, 1, C), lambda b, j: (b, 0, 0)),
            pl.BlockSpec((2 * C, C), lambda b, j: (0, 0)),
            pl.BlockSpec((C, C // 2), lambda b, j: (0, 0)),
            pl.BlockSpec((C // 2, 2), lambda b, j: (0, 0)),
            pl.BlockSpec(memory_space=pltpu.SMEM),
            pl.BlockSpec(memory_space=pltpu.SMEM),
            pl.BlockSpec(memory_space=pltpu.SMEM),
        ],
        out_specs=[
            pl.BlockSpec((1, 1, blk), lambda b, j: (b, 0, j)),
            pl.BlockSpec((1, 1, blk), lambda b, j: (b, 0, j)),
        ],
        out_shape=[
            jax.ShapeDtypeStruct((B, 1, HW), jnp.float32),
            jax.ShapeDtypeStruct((B, 1, HW), jnp.float32),
        ],
        compiler_params=pltpu.CompilerParams(
            dimension_semantics=("arbitrary", "arbitrary")),
    )(x3, mu2, sig2, lnw, lnb, wipT_bf, s1c, y2c, gxc, wcc1T_bf, wcc2T_bf,
      wcc3T_bf, wch_r, wc3_r, sig_a)
    return out, wgt


def kernel(x, mask, ratio, ln_w, ln_b, W_inproj, W_tc, W_cc1, W_cc2, W_cc3,
           W_ch, W_c3, alpha):
    B, C, H, W = x.shape
    HW = H * W
    blk = 1024
    x3 = x.reshape(B, C, HW)
    mask2 = mask.reshape(B, 1, HW)
    lnw = ln_w.reshape(1, C)
    lnb = ln_b.reshape(1, C)
    wipT_bf = W_inproj.T.astype(jnp.bfloat16)

    scale2 = jnp.einsum('bk,ok->bo', _rt_hard(ratio), _rt_hard(jnp.exp(W_tc)),
                        precision=jax.lax.Precision.HIGHEST)
    s1 = scale2[:, :C]
    s2 = scale2[:, C:]

    x3t = jnp.transpose(x3, (0, 2, 1))
    mu2 = x3t.mean(-1).reshape(B, 1, HW)
    sig2 = x3t.var(-1).reshape(B, 1, HW)

    num, den = _run_passA(x3, mu2, sig2, mask2, lnw, lnb, wipT_bf, blk)
    gx = num[:, 0, :] / den[:, :, 0]
    gx = jnp.where(gx == jnp.inf, 0.0, gx)

    s1c = s1[:, None, :]
    y2c = (gx * s2)[:, None, :]
    gxc = gx[:, None, :]

    rmax = ratio.max()
    a0 = alpha.reshape(())
    alpha_eff = jnp.where(a0 < rmax, rmax, a0)
    sig_a = jax.nn.sigmoid(alpha_eff).reshape(1, 1)

    out2, wgt2 = _run_passB(
        x3, mu2, sig2, lnw, lnb, wipT_bf, s1c, y2c, gxc,
        W_cc1.T.astype(jnp.bfloat16), W_cc2.T.astype(jnp.bfloat16),
        W_cc3.T.astype(jnp.bfloat16), _rt_hard(W_ch), _rt_hard(W_c3),
        sig_a, blk)

    output = out2.reshape(B, 1, 1, HW)
    weighting = wgt2.reshape(B, 1, H, W)

    m = rmax / alpha_eff
    K = jnp.where(m <= 0.4, m, jnp.float32(0.4))
    K = jnp.where(K <= 0.004, jnp.float32(0.005), K)
    kmax = (2 * HW) // 5
    k = jnp.floor(K * HW).astype(jnp.int32)
    _, idx_hi = jax.lax.top_k(output, kmax)
    _, idx_lo = jax.lax.top_k(-output, kmax)
    vals = jnp.broadcast_to((jnp.arange(kmax) < k).astype(jnp.float32),
                            (B, 1, 1, kmax))
    new_mask = jnp.zeros((B, 1, 1, HW), dtype=jnp.float32)
    bidx = jnp.arange(B).reshape(B, 1, 1, 1)
    new_mask = new_mask.at[bidx, 0, 0, idx_hi].max(vals)
    new_mask = new_mask.at[bidx, 0, 0, idx_lo].max(vals)
    return (new_mask, weighting)


# speedup vs baseline: 3.6763x; 3.4537x over previous
import jax
import jax.numpy as jnp
from jax.experimental import pallas as pl
from jax.experimental.pallas import tpu as pltpu

_INV_SQRT2 = 0.7071067811865476


def _gelu_exact(v):
    return 0.5 * v * (1.0 + jax.lax.erf(v * _INV_SQRT2))


def _rt(v):
    return v.astype(jnp.bfloat16).astype(jnp.float32)


def _rt_hard(v):
    b = jax.lax.bitcast_convert_type(v, jnp.uint32)
    r = b + jnp.uint32(0x7FFF) + ((b >> 16) & jnp.uint32(1))
    return jax.lax.bitcast_convert_type(r & jnp.uint32(0xFFFF0000),
                                        jnp.float32)


def _ln_inproj(x_ref, mu_ref, sig_ref, lnw_ref, lnb_ref, wipT_ref):
    xt = x_ref[0].T
    mu = mu_ref[0].T
    sig = sig_ref[0].T
    xn = (xt - mu) / jnp.sqrt(sig + 1e-5) * lnw_ref[...] + lnb_ref[...]
    local = _gelu_exact(
        jax.lax.dot(xn.astype(jnp.bfloat16), wipT_ref[...],
                    preferred_element_type=jnp.float32))
    return local


def _passA_body(x_ref, mu_ref, sig_ref, m_ref, lnw_ref, lnb_ref, wipT_ref,
                num_ref, den_ref):
    j = pl.program_id(1)
    local = _ln_inproj(x_ref, mu_ref, sig_ref, lnw_ref, lnb_ref, wipT_ref)
    mt = m_ref[0].T
    num_p = jnp.sum(local * mt, axis=0, keepdims=True)
    den_p = jnp.sum(mt)

    @pl.when(j == 0)
    def _():
        num_ref[...] = jnp.zeros_like(num_ref)
        den_ref[...] = jnp.zeros_like(den_ref)

    num_ref[...] += num_p[None]
    den_ref[...] += den_p.reshape(1, 1, 1)


def _passB_body(x_ref, mu_ref, sig_ref, lnw_ref, lnb_ref, wipT_ref,
                s1_ref, y2_ref, gx_ref, wcc1T_ref, wcc2T_ref, wcc3T_ref,
                wch_ref, wc3_ref, sa_ref, out_ref, wgt_ref):
    blk = x_ref.shape[2]
    C = x_ref.shape[1]
    local = _ln_inproj(x_ref, mu_ref, sig_ref, lnw_ref, lnb_ref, wipT_ref)
    gx = gx_ref[0]
    t = local * gx
    pmax = jnp.max(t, axis=-1, keepdims=True)
    pmean = t.mean(-1, keepdims=True)
    y1 = (local * s1_ref[0]).astype(jnp.bfloat16)
    y2 = jnp.broadcast_to(y2_ref[0].astype(jnp.bfloat16), (blk, C))
    y = jnp.concatenate([y1, y2], axis=1)
    z1 = _gelu_exact(jax.lax.dot(y, wcc1T_ref[...],
                                 preferred_element_type=jnp.float32))
    z2 = _gelu_exact(jax.lax.dot(z1.astype(jnp.bfloat16), wcc2T_ref[...],
                                 preferred_element_type=jnp.float32))
    z3 = _gelu_exact(jax.lax.dot(z2.astype(jnp.bfloat16), wcc3T_ref[...],
                                 preferred_element_type=jnp.float32))
    feat = jnp.concatenate([pmax, pmean, z3], axis=1)
    ft = feat.T
    x1_0 = _gelu_exact(wch_ref[0, 0] * _rt(ft[0:1]) + wch_ref[0, 1] * _rt(ft[1:2]))
    x1_1 = _gelu_exact(wch_ref[1, 0] * _rt(ft[0:1]) + wch_ref[1, 1] * _rt(ft[1:2]))
    pre = (wc3_ref[0, 0] * _rt(x1_0) + wc3_ref[0, 1] * _rt(x1_1)
           + wc3_ref[0, 2] * _rt(ft[2:3]) + wc3_ref[0, 3] * _rt(ft[3:4]))
    out = _gelu_exact(pre)
    out_ref[...] = out[None]
    wgt_ref[...] = (sa_ref[0, 0] * _gelu_exact(out))[None]


def _run_passA(x3, mu2, sig2, mask2, lnw, lnb, wipT_bf, blk):
    B, C, HW = x3.shape
    nb = HW // blk
    num, den = pl.pallas_call(
        _passA_body,
        grid=(B, nb),
        in_specs=[
            pl.BlockSpec((1, C, blk), lambda b, j: (b, 0, j)),
            pl.BlockSpec((1, 1, blk), lambda b, j: (b, 0, j)),
            pl.BlockSpec((1, 1, blk), lambda b, j: (b, 0, j)),
            pl.BlockSpec((1, 1, blk), lambda b, j: (b, 0, j)),
            pl.BlockSpec((1, C), lambda b, j: (0, 0)),
            pl.BlockSpec((1, C), lambda b, j: (0, 0)),
            pl.BlockSpec((C, C), lambda b, j: (0, 0)),
        ],
        out_specs=[
            pl.BlockSpec((1, 1, C), lambda b, j: (b, 0, 0)),
            pl.BlockSpec((1, 1, 1), lambda b, j: (b, 0, 0)),
        ],
        out_shape=[
            jax.ShapeDtypeStruct((B, 1, C), jnp.float32),
            jax.ShapeDtypeStruct((B, 1, 1), jnp.float32),
        ],
        compiler_params=pltpu.CompilerParams(
            dimension_semantics=("arbitrary", "arbitrary")),
    )(x3, mu2, sig2, mask2, lnw, lnb, wipT_bf)
    return num, den


def _run_passB(x3, mu2, sig2, lnw, lnb, wipT_bf, s1c, y2c, gxc, wcc1T_bf,
               wcc2T_bf, wcc3T_bf, wch_r, wc3_r, sig_a, blk):
    B, C, HW = x3.shape
    nb = HW // blk
    out, wgt = pl.pallas_call(
        _passB_body,
        grid=(B, nb),
        in_specs=[
            pl.BlockSpec((1, C, blk), lambda b, j: (b, 0, j)),
            pl.BlockSpec((1, 1, blk), lambda b, j: (b, 0, j)),
            pl.BlockSpec((1, 1, blk), lambda b, j: (b, 0, j)),
            pl.BlockSpec((1, C), lambda b, j: (0, 0)),
            pl.BlockSpec((1, C), lambda b, j: (0, 0)),
            pl.BlockSpec((C, C), lambda b, j: (0, 0)),
            pl.BlockSpec((1, 1, C), lambda b, j: (b, 0, 0)),
            pl.BlockSpec((1, 1, C), lambda b, j: (b, 0, 0)),
            pl.BlockSpec((1, 1, C), lambda b, j: (b, 0, 0)),
            pl.BlockSpec((2 * C, C), lambda b, j: (0, 0)),
            pl.BlockSpec((C, C // 2), lambda b, j: (0, 0)),
            pl.BlockSpec((C // 2, 2), lambda b, j: (0, 0)),
            pl.BlockSpec(memory_space=pltpu.SMEM),
            pl.BlockSpec(memory_space=pltpu.SMEM),
            pl.BlockSpec(memory_space=pltpu.SMEM),
        ],
        out_specs=[
            pl.BlockSpec((1, 1, blk), lambda b, j: (b, 0, j)),
            pl.BlockSpec((1, 1, blk), lambda b, j: (b, 0, j)),
        ],
        out_shape=[
            jax.ShapeDtypeStruct((B, 1, HW), jnp.float32),
            jax.ShapeDtypeStruct((B, 1, HW), jnp.float32),
        ],
        compiler_params=pltpu.CompilerParams(
            dimension_semantics=("arbitrary", "arbitrary")),
    )(x3, mu2, sig2, lnw, lnb, wipT_bf, s1c, y2c, gxc, wcc1T_bf, wcc2T_bf,
      wcc3T_bf, wch_r, wc3_r, sig_a)
    return out, wgt


def _mask_body(out_ref, kk_ref, mask_ref):
    HW = out_ref.shape[2]
    B = out_ref.shape[0]
    v = out_ref[:, 0, :]
    b = jax.lax.bitcast_convert_type(v, jnp.int32)
    ks_hi = b ^ (jax.lax.shift_right_arithmetic(b, 31) & jnp.int32(0x7FFFFFFF))
    idx = jax.lax.broadcasted_iota(jnp.int32, (B, HW), 1)
    kk = kk_ref[0, 0]
    full = jnp.zeros((B, HW), dtype=jnp.bool_)
    MIN32 = jnp.int32(-2147483648)
    for ks in (ks_hi, ~ks_hi):
        ub = ks ^ MIN32
        P = jnp.zeros((B, 1), dtype=jnp.int32)
        need = jnp.full((B, 1), kk, dtype=jnp.int32)
        for j in range(31, -1, -1):
            cand = P | (jnp.int32(1) << j)
            pj = jax.lax.shift_right_logical(cand, j)
            c1 = jnp.sum((jax.lax.shift_right_logical(ub, j) == pj)
                         .astype(jnp.int32), axis=1, keepdims=True)
            take = need <= c1
            P = jnp.where(take, cand, P)
            need = jnp.where(take, need, need - c1)
        eq = ub == P
        Pm = jnp.zeros((B, 1), dtype=jnp.int32)
        for j in range(15, -1, -1):
            c0 = jnp.sum((eq & (jax.lax.shift_right_logical(idx, j)
                                == jax.lax.shift_right_logical(Pm, j)))
                         .astype(jnp.int32), axis=1, keepdims=True)
            stay = need <= c0
            Pm = jnp.where(stay, Pm, Pm | (jnp.int32(1) << j))
            need = jnp.where(stay, need, need - c0)
        T_s = P ^ MIN32
        full = full | (ks > T_s) | (eq & (idx <= Pm))
    mask_ref[...] = full[:, None, :].astype(jnp.float32)


def _run_mask(out2, kk):
    B, _, HW = out2.shape
    return pl.pallas_call(
        _mask_body,
        grid=(1,),
        in_specs=[
            pl.BlockSpec((B, 1, HW), lambda i: (0, 0, 0)),
            pl.BlockSpec(memory_space=pltpu.SMEM),
        ],
        out_specs=pl.BlockSpec((B, 1, HW), lambda i: (0, 0, 0)),
        out_shape=jax.ShapeDtypeStruct((B, 1, HW), jnp.float32),
    )(out2, kk)


def kernel(x, mask, ratio, ln_w, ln_b, W_inproj, W_tc, W_cc1, W_cc2, W_cc3,
           W_ch, W_c3, alpha):
    B, C, H, W = x.shape
    HW = H * W
    blk = 1024
    x3 = x.reshape(B, C, HW)
    mask2 = mask.reshape(B, 1, HW)
    lnw = ln_w.reshape(1, C)
    lnb = ln_b.reshape(1, C)
    wipT_bf = W_inproj.T.astype(jnp.bfloat16)

    scale2 = jnp.einsum('bk,ok->bo', _rt_hard(ratio), _rt_hard(jnp.exp(W_tc)),
                        precision=jax.lax.Precision.HIGHEST)
    s1 = scale2[:, :C]
    s2 = scale2[:, C:]

    x3t = jnp.transpose(x3, (0, 2, 1))
    mu2 = x3t.mean(-1).reshape(B, 1, HW)
    sig2 = x3t.var(-1).reshape(B, 1, HW)

    num, den = _run_passA(x3, mu2, sig2, mask2, lnw, lnb, wipT_bf, blk)
    gx = num[:, 0, :] / den[:, :, 0]
    gx = jnp.where(gx == jnp.inf, 0.0, gx)

    s1c = s1[:, None, :]
    y2c = (gx * s2)[:, None, :]
    gxc = gx[:, None, :]

    rmax = ratio.max()
    a0 = alpha.reshape(())
    alpha_eff = jnp.where(a0 < rmax, rmax, a0)
    sig_a = jax.nn.sigmoid(alpha_eff).reshape(1, 1)

    out2, wgt2 = _run_passB(
        x3, mu2, sig2, lnw, lnb, wipT_bf, s1c, y2c, gxc,
        W_cc1.T.astype(jnp.bfloat16), W_cc2.T.astype(jnp.bfloat16),
        W_cc3.T.astype(jnp.bfloat16), _rt_hard(W_ch), _rt_hard(W_c3),
        sig_a, blk)

    output = out2.reshape(B, 1, 1, HW)
    weighting = wgt2.reshape(B, 1, H, W)

    m = rmax / alpha_eff
    K = jnp.where(m <= 0.4, m, jnp.float32(0.4))
    K = jnp.where(K <= 0.004, jnp.float32(0.005), K)
    k = jnp.floor(K * HW).astype(jnp.int32).reshape(1, 1)
    new_mask = _run_mask(out2, k).reshape(B, 1, 1, HW)
    return (new_mask, weighting)
